# baseline (speedup 1.0000x reference)
import dataclasses
import functools

import jax
import jax.numpy as jnp
from jax import lax
from jax.experimental import pallas as pl
from jax.experimental.pallas import tpu as pltpu
from jax.experimental.pallas import tpu_sc as plsc

N_NODES = 5000
D = 256
E = 160000

NC = 2
NS = 16
NP = 2
DP = D // NP
EPT = E // NS
CHUNK = 80
NCH = EPT // CHUNK
STRIPE = 312
TAIL = N_NODES - NS * STRIPE


def _sc_spmm(user_embedding, item_embedding, edge_user, edge_item,
             edge_weight):
    mesh = plsc.VectorSubcoreMesh(core_axis_name="c", subcore_axis_name="s")

    cp = pltpu.CompilerParams()
    cp = dataclasses.replace(cp, needs_layout_passes=False,
                             use_tc_tiling_on_sc=False)

    @functools.partial(
        pl.kernel,
        compiler_params=cp,
        out_type=jax.ShapeDtypeStruct((NC, NP, N_NODES, DP), jnp.float32),
        mesh=mesh,
        scratch_types=[
            pltpu.VMEM((CHUNK,), jnp.int32),
            pltpu.VMEM((CHUNK,), jnp.int32),
            pltpu.VMEM((CHUNK,), jnp.int32),
            pltpu.VMEM((CHUNK,), jnp.int32),
            pltpu.VMEM((CHUNK,), jnp.float32),
            pltpu.VMEM((CHUNK,), jnp.float32),
            pltpu.VMEM((CHUNK, DP), jnp.float32),
            pltpu.VMEM((CHUNK, DP), jnp.float32),
            pltpu.VMEM_SHARED((N_NODES, DP), jnp.float32),
            pltpu.VMEM_SHARED((N_NODES, DP), jnp.float32),
            pltpu.SemaphoreType.DMA,
            pltpu.SemaphoreType.DMA,
        ],
    )
    def k(ue_hbm, ie_hbm, eu_hbm, ei_hbm, w_hbm, out_hbm,
          gi_a, gi_b, si_a, si_b, w_a, w_b, rows_a, rows_b,
          table_sp, acc, gsem, esem):
        cid = lax.axis_index("c")
        sid = lax.axis_index("s")

        base = sid * STRIPE
        ebase = sid * EPT

        def scale(rows_v, w_v):
            @pl.loop(0, CHUNK)
            def _(r):
                wspl = plsc.load_gather(w_v, [jnp.full((16,), r, jnp.int32)])
                for f in range(DP // 16):
                    sl = (r, pl.ds(f * 16, 16))
                    rows_v.at[*sl][...] = rows_v.at[*sl][...] * wspl

        def stage_meta(j, gi_v, si_v, w_v):
            sl = pl.ds(ebase + j * CHUNK, CHUNK)
            @pl.when(cid == 0)
            def _():
                pltpu.async_copy(ei_hbm.at[sl], gi_v, esem)
                pltpu.async_copy(eu_hbm.at[sl], si_v, esem)

            @pl.when(cid == 1)
            def _():
                pltpu.async_copy(eu_hbm.at[sl], gi_v, esem)
                pltpu.async_copy(ei_hbm.at[sl], si_v, esem)
            pltpu.async_copy(w_hbm.at[sl], w_v, esem)

        def wait_meta(gi_v, si_v, w_v):
            sl = pl.ds(0, CHUNK)
            pltpu.make_async_copy(eu_hbm.at[sl], gi_v, esem).wait()
            pltpu.make_async_copy(eu_hbm.at[sl], si_v, esem).wait()
            pltpu.make_async_copy(w_hbm.at[sl], w_v, esem).wait()

        def gather(gi_v, rows_v):
            pltpu.async_copy(table_sp.at[gi_v], rows_v, gsem)

        def wait_gather(rows_v):
            pltpu.make_async_copy(table_sp.at[gi_a], rows_v, gsem).wait()

        def scatter_add(rows_v, si_v):
            pltpu.sync_copy(rows_v, acc.at[si_v], add=True)

        for p in range(NP):
            cols = pl.ds(p * DP, DP)

            @pl.when(cid == 0)
            def _():
                pltpu.sync_copy(ie_hbm.at[pl.ds(base, STRIPE), cols],
                                table_sp.at[pl.ds(base, STRIPE)])
                pltpu.sync_copy(ue_hbm.at[pl.ds(base, STRIPE), cols],
                                acc.at[pl.ds(base, STRIPE)])

                @pl.when(sid == 0)
                def _():
                    pltpu.sync_copy(ie_hbm.at[pl.ds(NS * STRIPE, TAIL), cols],
                                    table_sp.at[pl.ds(NS * STRIPE, TAIL)])
                    pltpu.sync_copy(ue_hbm.at[pl.ds(NS * STRIPE, TAIL), cols],
                                    acc.at[pl.ds(NS * STRIPE, TAIL)])

            @pl.when(cid == 1)
            def _():
                pltpu.sync_copy(ue_hbm.at[pl.ds(base, STRIPE), cols],
                                table_sp.at[pl.ds(base, STRIPE)])
                pltpu.sync_copy(ie_hbm.at[pl.ds(base, STRIPE), cols],
                                acc.at[pl.ds(base, STRIPE)])

                @pl.when(sid == 0)
                def _():
                    pltpu.sync_copy(ue_hbm.at[pl.ds(NS * STRIPE, TAIL), cols],
                                    table_sp.at[pl.ds(NS * STRIPE, TAIL)])
                    pltpu.sync_copy(ie_hbm.at[pl.ds(NS * STRIPE, TAIL), cols],
                                    acc.at[pl.ds(NS * STRIPE, TAIL)])

            plsc.subcore_barrier()

            stage_meta(0, gi_a, si_a, w_a)
            stage_meta(1, gi_b, si_b, w_b)
            wait_meta(gi_a, si_a, w_a)
            gather(gi_a, rows_a)
            wait_meta(gi_b, si_b, w_b)

            @pl.loop(0, NCH - 1, step=2)
            def _(j):
                wait_gather(rows_a)
                gather(gi_b, rows_b)
                scale(rows_a, w_a)
                scatter_add(rows_a, si_a)

                @pl.when(j + 2 < NCH)
                def _():
                    stage_meta(j + 2, gi_a, si_a, w_a)

                wait_gather(rows_b)

                @pl.when(j + 2 < NCH)
                def _():
                    wait_meta(gi_a, si_a, w_a)
                    gather(gi_a, rows_a)
                scale(rows_b, w_b)
                scatter_add(rows_b, si_b)

                @pl.when(j + 3 < NCH)
                def _():
                    stage_meta(j + 3, gi_b, si_b, w_b)
                    wait_meta(gi_b, si_b, w_b)

            wait_gather(rows_a)
            scale(rows_a, w_a)
            scatter_add(rows_a, si_a)

            plsc.subcore_barrier()

            pltpu.sync_copy(acc.at[pl.ds(base, STRIPE)],
                            out_hbm.at[cid, p, pl.ds(base, STRIPE)])

            @pl.when(sid == 0)
            def _():
                pltpu.sync_copy(acc.at[pl.ds(NS * STRIPE, TAIL)],
                                out_hbm.at[cid, p, pl.ds(NS * STRIPE, TAIL)])

    return k(user_embedding, item_embedding, edge_user, edge_item,
             edge_weight)


def _tc_dense(parts, u_w, i_w):
    BLK = 1000
    grid = (N_NODES // BLK,)

    def body(pu0, pu1, pi0, pi1, uw_ref, iw_ref, ou_ref, oi_ref):
        xu = jnp.concatenate([pu0[0, 0], pu1[0, 0]], axis=-1)
        ou_ref[...] = jax.nn.sigmoid(
            jnp.dot(xu, uw_ref[...], preferred_element_type=jnp.float32))
        xi = jnp.concatenate([pi0[0, 0], pi1[0, 0]], axis=-1)
        oi_ref[...] = jax.nn.sigmoid(
            jnp.dot(xi, iw_ref[...], preferred_element_type=jnp.float32))

    emb_spec = pl.BlockSpec((BLK, D), lambda i: (i, 0))

    def part_spec(c, p):
        return pl.BlockSpec((1, 1, BLK, DP), lambda i: (c, p, i, 0))

    w_spec = pl.BlockSpec((D, D), lambda i: (0, 0))

    return pl.pallas_call(
        body,
        grid=grid,
        in_specs=[part_spec(0, 0), part_spec(0, 1),
                  part_spec(1, 0), part_spec(1, 1), w_spec, w_spec],
        out_specs=[emb_spec, emb_spec],
        out_shape=[
            jax.ShapeDtypeStruct((N_NODES, D), jnp.float32),
            jax.ShapeDtypeStruct((N_NODES, D), jnp.float32),
        ],
    )(parts, parts, parts, parts, u_w, i_w)


def kernel(user_embedding, item_embedding, edge_user, edge_item, edge_weight,
           u_w, i_w, ind_beh):
    parts = _sc_spmm(user_embedding, item_embedding,
                     edge_user.astype(jnp.int32), edge_item.astype(jnp.int32),
                     edge_weight)
    u_emb, i_emb = _tc_dense(parts, u_w, i_w)
    return (u_emb, i_emb)


# speedup vs baseline: 1.3280x; 1.3280x over previous
import dataclasses
import functools

import jax
import jax.numpy as jnp
from jax import lax
from jax.experimental import pallas as pl
from jax.experimental.pallas import tpu as pltpu
from jax.experimental.pallas import tpu_sc as plsc

N_NODES = 5000
D = 256
E = 160000

NC = 2
NS = 16
EPT = E // NS
CHUNK = 80
NCH = EPT // CHUNK
STRIPE = 312
TAIL = N_NODES - NS * STRIPE


def _sc_spmm(user_embedding, item_embedding, edge_user, edge_item,
             edge_weight):
    mesh = plsc.VectorSubcoreMesh(core_axis_name="c", subcore_axis_name="s")

    cp = pltpu.CompilerParams()
    cp = dataclasses.replace(cp, needs_layout_passes=False,
                             use_tc_tiling_on_sc=False)

    @functools.partial(
        pl.kernel,
        compiler_params=cp,
        out_type=jax.ShapeDtypeStruct((NC, N_NODES, D), jnp.float32),
        mesh=mesh,
        scratch_types=[
            pltpu.VMEM((CHUNK,), jnp.int32),
            pltpu.VMEM((CHUNK,), jnp.int32),
            pltpu.VMEM((CHUNK,), jnp.int32),
            pltpu.VMEM((CHUNK,), jnp.int32),
            pltpu.VMEM((CHUNK,), jnp.float32),
            pltpu.VMEM((CHUNK,), jnp.float32),
            pltpu.VMEM((CHUNK, D), jnp.float32),
            pltpu.VMEM((CHUNK, D), jnp.float32),
            pltpu.VMEM_SHARED((N_NODES, D), jnp.float32),
            pltpu.SemaphoreType.DMA,
            pltpu.SemaphoreType.DMA,
        ],
    )
    def k(ue_hbm, ie_hbm, eu_hbm, ei_hbm, w_hbm, out_hbm,
          gi_a, gi_b, si_a, si_b, w_a, w_b, rows_a, rows_b, acc, gsem, esem):
        cid = lax.axis_index("c")
        sid = lax.axis_index("s")

        base = sid * STRIPE

        @pl.when(cid == 0)
        def _():
            pltpu.sync_copy(ue_hbm.at[pl.ds(base, STRIPE)],
                            acc.at[pl.ds(base, STRIPE)])

            @pl.when(sid == 0)
            def _():
                pltpu.sync_copy(ue_hbm.at[pl.ds(NS * STRIPE, TAIL)],
                                acc.at[pl.ds(NS * STRIPE, TAIL)])

        @pl.when(cid == 1)
        def _():
            pltpu.sync_copy(ie_hbm.at[pl.ds(base, STRIPE)],
                            acc.at[pl.ds(base, STRIPE)])

            @pl.when(sid == 0)
            def _():
                pltpu.sync_copy(ie_hbm.at[pl.ds(NS * STRIPE, TAIL)],
                                acc.at[pl.ds(NS * STRIPE, TAIL)])

        plsc.subcore_barrier()

        ebase = sid * EPT

        def scale(rows_v, w_v):
            @pl.loop(0, CHUNK)
            def _(r):
                wspl = plsc.load_gather(w_v, [jnp.full((16,), r, jnp.int32)])
                for f in range(D // 16):
                    sl = (r, pl.ds(f * 16, 16))
                    rows_v.at[*sl][...] = rows_v.at[*sl][...] * wspl

        def stage_meta(j, gi_v, si_v, w_v):
            sl = pl.ds(ebase + j * CHUNK, CHUNK)
            @pl.when(cid == 0)
            def _():
                pltpu.async_copy(ei_hbm.at[sl], gi_v, esem)
                pltpu.async_copy(eu_hbm.at[sl], si_v, esem)

            @pl.when(cid == 1)
            def _():
                pltpu.async_copy(eu_hbm.at[sl], gi_v, esem)
                pltpu.async_copy(ei_hbm.at[sl], si_v, esem)
            pltpu.async_copy(w_hbm.at[sl], w_v, esem)

        def wait_meta(gi_v, si_v, w_v):
            sl = pl.ds(0, CHUNK)
            pltpu.make_async_copy(eu_hbm.at[sl], gi_v, esem).wait()
            pltpu.make_async_copy(eu_hbm.at[sl], si_v, esem).wait()
            pltpu.make_async_copy(w_hbm.at[sl], w_v, esem).wait()

        def gather(gi_v, rows_v):
            @pl.when(cid == 0)
            def _():
                pltpu.async_copy(ie_hbm.at[gi_v], rows_v, gsem)

            @pl.when(cid == 1)
            def _():
                pltpu.async_copy(ue_hbm.at[gi_v], rows_v, gsem)

        def wait_gather(rows_v):
            pltpu.make_async_copy(ie_hbm.at[gi_a], rows_v, gsem).wait()

        def scatter_add(rows_v, si_v):
            pltpu.sync_copy(rows_v, acc.at[si_v], add=True)

        stage_meta(0, gi_a, si_a, w_a)
        stage_meta(1, gi_b, si_b, w_b)
        wait_meta(gi_a, si_a, w_a)
        gather(gi_a, rows_a)
        wait_meta(gi_b, si_b, w_b)

        @pl.loop(0, NCH - 1, step=2)
        def _(j):
            wait_gather(rows_a)
            gather(gi_b, rows_b)
            scale(rows_a, w_a)
            scatter_add(rows_a, si_a)

            @pl.when(j + 2 < NCH)
            def _():
                stage_meta(j + 2, gi_a, si_a, w_a)

            wait_gather(rows_b)

            @pl.when(j + 2 < NCH)
            def _():
                wait_meta(gi_a, si_a, w_a)
                gather(gi_a, rows_a)
            scale(rows_b, w_b)
            scatter_add(rows_b, si_b)

            @pl.when(j + 3 < NCH)
            def _():
                stage_meta(j + 3, gi_b, si_b, w_b)
                wait_meta(gi_b, si_b, w_b)

        wait_gather(rows_a)
        scale(rows_a, w_a)
        scatter_add(rows_a, si_a)

        plsc.subcore_barrier()

        pltpu.sync_copy(acc.at[pl.ds(base, STRIPE)],
                        out_hbm.at[cid, pl.ds(base, STRIPE)])

        @pl.when(sid == 0)
        def _():
            pltpu.sync_copy(acc.at[pl.ds(NS * STRIPE, TAIL)],
                            out_hbm.at[cid, pl.ds(NS * STRIPE, TAIL)])

    return k(user_embedding, item_embedding, edge_user, edge_item,
             edge_weight)


def _tc_dense(parts, u_w, i_w):
    BLK = 1000
    grid = (N_NODES // BLK,)

    def body(pu_ref, pi_ref, uw_ref, iw_ref, ou_ref, oi_ref):
        ou_ref[...] = jax.nn.sigmoid(
            jnp.dot(pu_ref[0], uw_ref[...], preferred_element_type=jnp.float32))
        oi_ref[...] = jax.nn.sigmoid(
            jnp.dot(pi_ref[0], iw_ref[...], preferred_element_type=jnp.float32))

    emb_spec = pl.BlockSpec((BLK, D), lambda i: (i, 0))
    pu_spec = pl.BlockSpec((1, BLK, D), lambda i: (0, i, 0))
    pi_spec = pl.BlockSpec((1, BLK, D), lambda i: (1, i, 0))
    w_spec = pl.BlockSpec((D, D), lambda i: (0, 0))

    return pl.pallas_call(
        body,
        grid=grid,
        in_specs=[pu_spec, pi_spec, w_spec, w_spec],
        out_specs=[emb_spec, emb_spec],
        out_shape=[
            jax.ShapeDtypeStruct((N_NODES, D), jnp.float32),
            jax.ShapeDtypeStruct((N_NODES, D), jnp.float32),
        ],
    )(parts, parts, u_w, i_w)


def kernel(user_embedding, item_embedding, edge_user, edge_item, edge_weight,
           u_w, i_w, ind_beh):
    parts = _sc_spmm(user_embedding, item_embedding,
                     edge_user.astype(jnp.int32), edge_item.astype(jnp.int32),
                     edge_weight)
    u_emb, i_emb = _tc_dense(parts, u_w, i_w)
    return (u_emb, i_emb)
